# baseline (speedup 1.0000x reference)
import jax
import jax.numpy as jnp
from jax import lax
from jax.experimental import pallas as pl
from jax.experimental.pallas import tpu as pltpu
from jax.experimental.pallas import tpu_sc as plsc

N = 10000
E = 160000
NC, NS = 2, 16
NW = NC * NS
B = 64
NB = 80
E_PAD = NW * NB * B
TRASH = N
N_ACC = 10240
RPT = N_ACC // NS
NP = RPT // B
CW = 128


def _sc_agg(C, with_deg):
  mesh = plsc.VectorSubcoreMesh(
      core_axis_name="c", subcore_axis_name="s", num_cores=NC, num_subcores=NS
  )
  CT = C + 1 if with_deg else C
  out_type = [jax.ShapeDtypeStruct((NC, CT, N_ACC, CW), jnp.float32)]
  scratch = [
      pltpu.VMEM((NB, B), jnp.int32),
      pltpu.VMEM((NB, B), jnp.int32),
      pltpu.VMEM((B, CW), jnp.float32),
      pltpu.VMEM((B, CW), jnp.float32),
      pltpu.VMEM_SHARED((N_ACC, CW), jnp.float32),
      pltpu.SemaphoreType.DMA,
      pltpu.SemaphoreType.DMA,
      pltpu.SemaphoreType.DMA,
  ]

  def body(xc, srcr, dstr, outp, src_v, dst_v, rows0, rows1, acc, s0, s1, t0):
    cid = lax.axis_index("c")
    sid = lax.axis_index("s")
    wid = cid * NS + sid
    pltpu.sync_copy(srcr.at[wid], src_v)
    pltpu.sync_copy(dstr.at[wid], dst_v)

    def fill(val):
      v = jnp.full((16,), val, jnp.float32)

      def row(i, _):
        for j in range(CW // 16):
          rows0[i, pl.ds(j * 16, 16)] = v
        return 0

      lax.fori_loop(0, B, row, 0)

    for c in range(CT):
      is_deg = with_deg and c == C
      fill(0.0)
      for p in range(NP):
        pltpu.sync_copy(rows0, acc.at[pl.ds(sid * RPT + p * B, B)])
      plsc.subcore_barrier()

      if is_deg:
        fill(1.0)

        def dbatch(b, _):
          pltpu.async_copy(rows0, acc.at[dst_v.at[b]], t0, add=True)
          return 0

        lax.fori_loop(0, NB, dbatch, 0)

        def dwait(b, _):
          pltpu.make_async_copy(rows0, acc.at[dst_v.at[0]], t0).wait()
          return 0

        lax.fori_loop(0, NB, dwait, 0)
      else:
        xcc = xc.at[c]
        pltpu.async_copy(xcc.at[src_v.at[0]], rows0, s0)

        def pair(g, _):
          b0 = g * 2
          pltpu.async_copy(xcc.at[src_v.at[b0 + 1]], rows1, s1)
          pltpu.make_async_copy(xcc.at[src_v.at[b0]], rows0, s0).wait()
          pltpu.sync_copy(rows0, acc.at[dst_v.at[b0]], add=True)

          @pl.when(g < NB // 2 - 1)
          def _():
            pltpu.async_copy(xcc.at[src_v.at[b0 + 2]], rows0, s0)

          pltpu.make_async_copy(xcc.at[src_v.at[b0 + 1]], rows1, s1).wait()
          pltpu.sync_copy(rows1, acc.at[dst_v.at[b0 + 1]], add=True)
          return 0

        lax.fori_loop(0, NB // 2, pair, 0)
      plsc.subcore_barrier()
      for p in range(NP):
        pltpu.sync_copy(acc.at[pl.ds(sid * RPT + p * B, B)], rows0)
        pltpu.sync_copy(rows0, outp.at[cid, c, pl.ds(sid * RPT + p * B, B)])
      plsc.subcore_barrier()

  return pl.kernel(body, out_type=out_type, mesh=mesh, scratch_types=scratch)


_sc_agg1 = _sc_agg(2, True)
_sc_agg2 = _sc_agg(4, False)

MB = 400
G = N // MB


def _tc1_body(pa, x, wl1, wr1, b1, wl2, wr2, b2, hwl2, outp):
  agg = jnp.concatenate([pa[0, 0] + pa[1, 0], pa[0, 1] + pa[1, 1]], axis=1)
  deg = pa[0, 2, :, 0:1] + pa[1, 2, :, 0:1]
  agg = agg / jnp.maximum(deg, 1.0)
  h = agg @ wl1[...] + x[...] @ wr1[...] + b1[...]
  h = jnp.maximum(h, 0.0)
  o1 = h @ wl2[...]
  for c in range(4):
    hwl2[c] = o1[:, c * CW:(c + 1) * CW]
  outp[...] = h @ wr2[...] + b2[...]


_tc1 = pl.pallas_call(
    _tc1_body,
    grid=(G,),
    in_specs=[
        pl.BlockSpec((NC, 3, MB, CW), lambda m: (0, 0, m, 0)),
        pl.BlockSpec((MB, 256), lambda m: (m, 0)),
        pl.BlockSpec((256, 1024), lambda m: (0, 0)),
        pl.BlockSpec((256, 1024), lambda m: (0, 0)),
        pl.BlockSpec((1, 1024), lambda m: (0, 0)),
        pl.BlockSpec((1024, 512), lambda m: (0, 0)),
        pl.BlockSpec((1024, 512), lambda m: (0, 0)),
        pl.BlockSpec((1, 512), lambda m: (0, 0)),
    ],
    out_specs=[
        pl.BlockSpec((4, MB, CW), lambda m: (0, m, 0)),
        pl.BlockSpec((MB, 512), lambda m: (m, 0)),
    ],
    out_shape=[
        jax.ShapeDtypeStruct((4, N, CW), jnp.float32),
        jax.ShapeDtypeStruct((N, 512), jnp.float32),
    ],
)


def _tc2_body(pb, pdeg, op, out):
  s = pb[0] + pb[1]
  agg = jnp.concatenate([s[c] for c in range(4)], axis=1)
  deg = pdeg[0, 0, :, 0:1] + pdeg[1, 0, :, 0:1]
  out[...] = agg / jnp.maximum(deg, 1.0) + op[...]


_tc2 = pl.pallas_call(
    _tc2_body,
    grid=(G,),
    in_specs=[
        pl.BlockSpec((NC, 4, MB, CW), lambda m: (0, 0, m, 0)),
        pl.BlockSpec((NC, 1, MB, CW), lambda m: (0, 2, m, 0)),
        pl.BlockSpec((MB, 512), lambda m: (m, 0)),
    ],
    out_specs=pl.BlockSpec((MB, 512), lambda m: (m, 0)),
    out_shape=jax.ShapeDtypeStruct((N, 512), jnp.float32),
)


@jax.jit
def kernel(x, edge_index, Wl1, Wr1, b1, Wl2, Wr2, b2):
  pad = E_PAD - E
  ar = jnp.arange(pad, dtype=jnp.int32)
  src = jnp.concatenate([edge_index[0], ar % N])
  dst = jnp.concatenate([edge_index[1], TRASH + ar % (N_ACC - N)])
  src = src.reshape(NW, NB, B)
  dst = dst.reshape(NW, NB, B)
  xc = jnp.stack([x[:, :CW], x[:, CW:]])

  (pa,) = _sc_agg1(xc, src, dst)
  hwl2, outpart = _tc1(pa, x, Wl1, Wr1,
                       b1.reshape(1, -1), Wl2, Wr2, b2.reshape(1, -1))
  (pb,) = _sc_agg2(hwl2, src, dst)
  return _tc2(pb, pa, outpart)


# speedup vs baseline: 1.0451x; 1.0451x over previous
import jax
import jax.numpy as jnp
from jax import lax
from jax.experimental import pallas as pl
from jax.experimental.pallas import tpu as pltpu
from jax.experimental.pallas import tpu_sc as plsc

N = 10000
E = 160000
NC, NS = 2, 16
NW = NC * NS
B = 64
NB = 80
E_PAD = NW * NB * B
TRASH = N
N_ACC = 10240
RPT = N_ACC // NS
NP = RPT // B
CW = 128


def _sc_agg(C, with_deg):
  mesh = plsc.VectorSubcoreMesh(
      core_axis_name="c", subcore_axis_name="s", num_cores=NC, num_subcores=NS
  )
  CT = C + 1 if with_deg else C
  out_type = [jax.ShapeDtypeStruct((NC, CT, N_ACC, CW), jnp.float32)]
  scratch = [
      pltpu.VMEM((NB, B), jnp.int32),
      pltpu.VMEM((NB, B), jnp.int32),
      pltpu.VMEM((B, CW), jnp.float32),
      pltpu.VMEM((B, CW), jnp.float32),
      pltpu.VMEM((B, CW), jnp.float32),
      pltpu.VMEM_SHARED((N_ACC, CW), jnp.float32),
      pltpu.SemaphoreType.DMA,
      pltpu.SemaphoreType.DMA,
      pltpu.SemaphoreType.DMA,
      pltpu.SemaphoreType.DMA,
      pltpu.SemaphoreType.DMA,
  ]

  def body(xc, srcr, dstr, outp, src_v, dst_v, rows0, rows1, zbuf, acc,
           s0, s1, t0, t1, t2):
    cid = lax.axis_index("c")
    sid = lax.axis_index("s")
    wid = cid * NS + sid
    pltpu.sync_copy(srcr.at[wid], src_v)
    pltpu.sync_copy(dstr.at[wid], dst_v)

    def fill(val):
      v = jnp.full((16,), val, jnp.float32)

      def row(i, _):
        for j in range(CW // 16):
          zbuf[i, pl.ds(j * 16, 16)] = v
        return 0

      lax.fori_loop(0, B, row, 0)

    def piece(p):
      return pl.ds(sid * RPT + p * B, B)

    fill(0.0)
    for p in range(NP):
      pltpu.async_copy(zbuf, acc.at[piece(p)], t2)
    for p in range(NP):
      pltpu.make_async_copy(zbuf, acc.at[piece(0)], t2).wait()
    plsc.subcore_barrier()

    for c in range(CT):
      is_deg = with_deg and c == C
      if is_deg:
        fill(1.0)

        def dbatch(b, _):
          pltpu.async_copy(zbuf, acc.at[dst_v.at[b]], t0, add=True)
          return 0

        lax.fori_loop(0, NB, dbatch, 0)

        def dwait(b, _):
          pltpu.make_async_copy(zbuf, acc.at[dst_v.at[0]], t0).wait()
          return 0

        lax.fori_loop(0, NB, dwait, 0)
        fill(0.0)
      else:
        xcc = xc.at[c]
        pltpu.async_copy(xcc.at[src_v.at[0]], rows0, s0)

        def pair(g, _):
          b0 = g * 2
          pltpu.async_copy(xcc.at[src_v.at[b0 + 1]], rows1, s1)
          pltpu.make_async_copy(xcc.at[src_v.at[b0]], rows0, s0).wait()
          pltpu.sync_copy(rows0, acc.at[dst_v.at[b0]], add=True)

          @pl.when(g < NB // 2 - 1)
          def _():
            pltpu.async_copy(xcc.at[src_v.at[b0 + 2]], rows0, s0)

          pltpu.make_async_copy(xcc.at[src_v.at[b0 + 1]], rows1, s1).wait()
          pltpu.sync_copy(rows1, acc.at[dst_v.at[b0 + 1]], add=True)
          return 0

        lax.fori_loop(0, NB // 2, pair, 0)
      plsc.subcore_barrier()
      for p in range(NP):
        r, tp = (rows0, t0) if p % 2 == 0 else (rows1, t1)
        if p >= 2:
          pltpu.make_async_copy(r, outp.at[cid, c, piece(0)], tp).wait()
        pltpu.sync_copy(acc.at[piece(p)], r)
        pltpu.async_copy(r, outp.at[cid, c, piece(p)], tp)
        if c < CT - 1:
          pltpu.async_copy(zbuf, acc.at[piece(p)], t2)
      for p in range(max(0, NP - 2), NP):
        r, tp = (rows0, t0) if p % 2 == 0 else (rows1, t1)
        pltpu.make_async_copy(r, outp.at[cid, c, piece(0)], tp).wait()
      if c < CT - 1:
        for p in range(NP):
          pltpu.make_async_copy(zbuf, acc.at[piece(0)], t2).wait()
      plsc.subcore_barrier()

  return pl.kernel(body, out_type=out_type, mesh=mesh, scratch_types=scratch)


_sc_agg1 = _sc_agg(2, True)
_sc_agg2 = _sc_agg(4, False)

MB = 400
G = N // MB


def _tc1_body(pa, x, wl1, wr1, b1, wl2, wr2, b2, hwl2, outp):
  agg = jnp.concatenate([pa[0, 0] + pa[1, 0], pa[0, 1] + pa[1, 1]], axis=1)
  deg = pa[0, 2, :, 0:1] + pa[1, 2, :, 0:1]
  agg = agg / jnp.maximum(deg, 1.0)
  h = agg @ wl1[...] + x[...] @ wr1[...] + b1[...]
  h = jnp.maximum(h, 0.0)
  o1 = h @ wl2[...]
  for c in range(4):
    hwl2[c] = o1[:, c * CW:(c + 1) * CW]
  outp[...] = h @ wr2[...] + b2[...]


_tc1 = pl.pallas_call(
    _tc1_body,
    grid=(G,),
    in_specs=[
        pl.BlockSpec((NC, 3, MB, CW), lambda m: (0, 0, m, 0)),
        pl.BlockSpec((MB, 256), lambda m: (m, 0)),
        pl.BlockSpec((256, 1024), lambda m: (0, 0)),
        pl.BlockSpec((256, 1024), lambda m: (0, 0)),
        pl.BlockSpec((1, 1024), lambda m: (0, 0)),
        pl.BlockSpec((1024, 512), lambda m: (0, 0)),
        pl.BlockSpec((1024, 512), lambda m: (0, 0)),
        pl.BlockSpec((1, 512), lambda m: (0, 0)),
    ],
    out_specs=[
        pl.BlockSpec((4, MB, CW), lambda m: (0, m, 0)),
        pl.BlockSpec((MB, 512), lambda m: (m, 0)),
    ],
    out_shape=[
        jax.ShapeDtypeStruct((4, N, CW), jnp.float32),
        jax.ShapeDtypeStruct((N, 512), jnp.float32),
    ],
)


def _tc2_body(pb, pdeg, op, out):
  s = pb[0] + pb[1]
  agg = jnp.concatenate([s[c] for c in range(4)], axis=1)
  deg = pdeg[0, 0, :, 0:1] + pdeg[1, 0, :, 0:1]
  out[...] = agg / jnp.maximum(deg, 1.0) + op[...]


_tc2 = pl.pallas_call(
    _tc2_body,
    grid=(G,),
    in_specs=[
        pl.BlockSpec((NC, 4, MB, CW), lambda m: (0, 0, m, 0)),
        pl.BlockSpec((NC, 1, MB, CW), lambda m: (0, 2, m, 0)),
        pl.BlockSpec((MB, 512), lambda m: (m, 0)),
    ],
    out_specs=pl.BlockSpec((MB, 512), lambda m: (m, 0)),
    out_shape=jax.ShapeDtypeStruct((N, 512), jnp.float32),
)


@jax.jit
def kernel(x, edge_index, Wl1, Wr1, b1, Wl2, Wr2, b2):
  pad = E_PAD - E
  ar = jnp.arange(pad, dtype=jnp.int32)
  src = jnp.concatenate([edge_index[0], ar % N])
  dst = jnp.concatenate([edge_index[1], TRASH + ar % (N_ACC - N)])
  src = src.reshape(NW, NB, B)
  dst = dst.reshape(NW, NB, B)
  xc = jnp.stack([x[:, :CW], x[:, CW:]])

  (pa,) = _sc_agg1(xc, src, dst)
  hwl2, outpart = _tc1(pa, x, Wl1, Wr1,
                       b1.reshape(1, -1), Wl2, Wr2, b2.reshape(1, -1))
  (pb,) = _sc_agg2(hwl2, src, dst)
  return _tc2(pb, pa, outpart)


# speedup vs baseline: 1.0504x; 1.0050x over previous
import jax
import jax.numpy as jnp
from jax import lax
from jax.experimental import pallas as pl
from jax.experimental.pallas import tpu as pltpu
from jax.experimental.pallas import tpu_sc as plsc

N = 10000
E = 160000
NC, NS = 2, 16
NW = NC * NS
B = 64
NB = 80
E_PAD = NW * NB * B
TRASH = N
N_ACC = 10240
RPT = N_ACC // NS
NP = RPT // B
CW = 128


def _sc_agg(C, with_deg):
  mesh = plsc.VectorSubcoreMesh(
      core_axis_name="c", subcore_axis_name="s", num_cores=NC, num_subcores=NS
  )
  CT = C + 1 if with_deg else C
  out_type = [jax.ShapeDtypeStruct((NC, CT, N_ACC, CW), jnp.float32)]
  scratch = [
      pltpu.VMEM((NB, B), jnp.int32),
      pltpu.VMEM((NB, B), jnp.int32),
      pltpu.VMEM((B, CW), jnp.float32),
      pltpu.VMEM((B, CW), jnp.float32),
      pltpu.VMEM((B, CW), jnp.float32),
      pltpu.VMEM_SHARED((N_ACC, CW), jnp.float32),
      pltpu.SemaphoreType.DMA,
      pltpu.SemaphoreType.DMA,
      pltpu.SemaphoreType.DMA,
      pltpu.SemaphoreType.DMA,
      pltpu.SemaphoreType.DMA,
  ]

  def body(xc, srcr, dstr, outp, src_v, dst_v, rows0, rows1, zbuf, acc,
           s0, s1, t0, t1, t2):
    cid = lax.axis_index("c")
    sid = lax.axis_index("s")
    wid = cid * NS + sid
    pltpu.sync_copy(srcr.at[wid], src_v)
    pltpu.sync_copy(dstr.at[wid], dst_v)

    def fill(val):
      v = jnp.full((16,), val, jnp.float32)

      def row(i, _):
        for j in range(CW // 16):
          zbuf[i, pl.ds(j * 16, 16)] = v
        return 0

      lax.fori_loop(0, B, row, 0)

    def piece(p):
      return pl.ds(sid * RPT + p * B, B)

    fill(0.0)
    for p in range(NP):
      pltpu.async_copy(zbuf, acc.at[piece(p)], t2)
    for p in range(NP):
      pltpu.make_async_copy(zbuf, acc.at[piece(0)], t2).wait()
    plsc.subcore_barrier()

    for c in range(CT):
      is_deg = with_deg and c == C
      if is_deg:
        fill(1.0)

        def dbatch(b, _):
          pltpu.async_copy(zbuf, acc.at[dst_v.at[b]], t0, add=True)
          return 0

        lax.fori_loop(0, NB, dbatch, 0)

        def dwait(b, _):
          pltpu.make_async_copy(zbuf, acc.at[dst_v.at[0]], t0).wait()
          return 0

        lax.fori_loop(0, NB, dwait, 0)
        fill(0.0)
      else:
        xcc = xc.at[c]
        pltpu.async_copy(xcc.at[src_v.at[0]], rows0, s0)

        def pair(g, _):
          b0 = g * 2
          pltpu.async_copy(xcc.at[src_v.at[b0 + 1]], rows1, s1)
          pltpu.make_async_copy(xcc.at[src_v.at[b0]], rows0, s0).wait()
          pltpu.sync_copy(rows0, acc.at[dst_v.at[b0]], add=True)

          @pl.when(g < NB // 2 - 1)
          def _():
            pltpu.async_copy(xcc.at[src_v.at[b0 + 2]], rows0, s0)

          pltpu.make_async_copy(xcc.at[src_v.at[b0 + 1]], rows1, s1).wait()
          pltpu.sync_copy(rows1, acc.at[dst_v.at[b0 + 1]], add=True)
          return 0

        lax.fori_loop(0, NB // 2, pair, 0)
      plsc.subcore_barrier()
      for p in range(NP):
        r, tp = (rows0, t0) if p % 2 == 0 else (rows1, t1)
        if p >= 2:
          pltpu.make_async_copy(r, outp.at[cid, c, piece(0)], tp).wait()
        pltpu.sync_copy(acc.at[piece(p)], r)
        pltpu.async_copy(r, outp.at[cid, c, piece(p)], tp)
        if c < CT - 1:
          pltpu.async_copy(zbuf, acc.at[piece(p)], t2)
      for p in range(max(0, NP - 2), NP):
        r, tp = (rows0, t0) if p % 2 == 0 else (rows1, t1)
        pltpu.make_async_copy(r, outp.at[cid, c, piece(0)], tp).wait()
      if c < CT - 1:
        for p in range(NP):
          pltpu.make_async_copy(zbuf, acc.at[piece(0)], t2).wait()
      plsc.subcore_barrier()

  return pl.kernel(body, out_type=out_type, mesh=mesh, scratch_types=scratch)


_sc_agg1 = _sc_agg(2, True)
_sc_agg2 = _sc_agg(4, False)

MB = 1000
G = N // MB


def _tc0_body(x, wr1, b1, xwr1b):
  xwr1b[...] = x[...] @ wr1[...] + b1[...]


_tc0 = pl.pallas_call(
    _tc0_body,
    grid=(G,),
    in_specs=[
        pl.BlockSpec((MB, 256), lambda m: (m, 0)),
        pl.BlockSpec((256, 1024), lambda m: (0, 0)),
        pl.BlockSpec((1, 1024), lambda m: (0, 0)),
    ],
    out_specs=pl.BlockSpec((MB, 1024), lambda m: (m, 0)),
    out_shape=jax.ShapeDtypeStruct((N, 1024), jnp.float32),
)


def _tc1_body(pa, xwr1b, wl1, wl2, wr2, b2, hwl2, outp):
  agg = jnp.concatenate([pa[0, 0] + pa[1, 0], pa[0, 1] + pa[1, 1]], axis=1)
  deg = pa[0, 2, :, 0:1] + pa[1, 2, :, 0:1]
  agg = agg / jnp.maximum(deg, 1.0)
  h = agg @ wl1[...] + xwr1b[...]
  h = jnp.maximum(h, 0.0)
  o1 = h @ wl2[...]
  for c in range(4):
    hwl2[c] = o1[:, c * CW:(c + 1) * CW]
  outp[...] = h @ wr2[...] + b2[...]


_tc1 = pl.pallas_call(
    _tc1_body,
    grid=(G,),
    in_specs=[
        pl.BlockSpec((NC, 3, MB, CW), lambda m: (0, 0, m, 0)),
        pl.BlockSpec((MB, 1024), lambda m: (m, 0)),
        pl.BlockSpec((256, 1024), lambda m: (0, 0)),
        pl.BlockSpec((1024, 512), lambda m: (0, 0)),
        pl.BlockSpec((1024, 512), lambda m: (0, 0)),
        pl.BlockSpec((1, 512), lambda m: (0, 0)),
    ],
    out_specs=[
        pl.BlockSpec((4, MB, CW), lambda m: (0, m, 0)),
        pl.BlockSpec((MB, 512), lambda m: (m, 0)),
    ],
    out_shape=[
        jax.ShapeDtypeStruct((4, N, CW), jnp.float32),
        jax.ShapeDtypeStruct((N, 512), jnp.float32),
    ],
)


def _tc2_body(pb, pdeg, op, out):
  s = pb[0] + pb[1]
  agg = jnp.concatenate([s[c] for c in range(4)], axis=1)
  deg = pdeg[0, 0, :, 0:1] + pdeg[1, 0, :, 0:1]
  out[...] = agg / jnp.maximum(deg, 1.0) + op[...]


_tc2 = pl.pallas_call(
    _tc2_body,
    grid=(G,),
    in_specs=[
        pl.BlockSpec((NC, 4, MB, CW), lambda m: (0, 0, m, 0)),
        pl.BlockSpec((NC, 1, MB, CW), lambda m: (0, 2, m, 0)),
        pl.BlockSpec((MB, 512), lambda m: (m, 0)),
    ],
    out_specs=pl.BlockSpec((MB, 512), lambda m: (m, 0)),
    out_shape=jax.ShapeDtypeStruct((N, 512), jnp.float32),
)


@jax.jit
def kernel(x, edge_index, Wl1, Wr1, b1, Wl2, Wr2, b2):
  pad = E_PAD - E
  ar = jnp.arange(pad, dtype=jnp.int32)
  src = jnp.concatenate([edge_index[0], ar % N])
  dst = jnp.concatenate([edge_index[1], TRASH + ar % (N_ACC - N)])
  src = src.reshape(NW, NB, B)
  dst = dst.reshape(NW, NB, B)
  xc = jnp.stack([x[:, :CW], x[:, CW:]])

  xwr1b = _tc0(x, Wr1, b1.reshape(1, -1))
  (pa,) = _sc_agg1(xc, src, dst)
  hwl2, outpart = _tc1(pa, xwr1b, Wl1, Wl2, Wr2, b2.reshape(1, -1))
  (pb,) = _sc_agg2(hwl2, src, dst)
  return _tc2(pb, pa, outpart)
